# baseline (speedup 1.0000x reference)
import functools

import jax
import jax.numpy as jnp
from jax.experimental import pallas as pl
from jax.experimental.pallas import tpu as pltpu

_PARALLEL = pltpu.CompilerParams(dimension_semantics=("parallel",))

K1 = 33
EPS = 0.3
FUSION = 0.1

_HIGH = jax.lax.Precision.DEFAULT
_INT_MIN = -2147483648


def _linear_kernel(x_ref, w_ref, b_ref, o_ref):
    o_ref[...] = (
        jnp.dot(x_ref[...], w_ref[...], precision=_HIGH,
                preferred_element_type=jnp.float32)
        + b_ref[...]
    )


def _linear(x, w, b):
    n, d = x.shape
    return pl.pallas_call(
        _linear_kernel,
        out_shape=jax.ShapeDtypeStruct((n, d), jnp.float32),
    )(x, w, b.reshape(1, d))


def _adj_gemm_kernel(adj_ref, y_ref, o_ref, *, relu):
    acc = jax.lax.dot_general(
        adj_ref[...], y_ref[...], (((1,), (0,)), ((), ())),
        precision=_HIGH, preferred_element_type=jnp.float32)
    o_ref[...] = jnp.maximum(acc, 0.0) if relu else acc


def _adj_gemm(adj, y, relu, blk):
    n, d = y.shape
    return pl.pallas_call(
        functools.partial(_adj_gemm_kernel, relu=relu),
        grid=(n // blk,),
        in_specs=[
            pl.BlockSpec((blk, n), lambda i: (i, 0)),
            pl.BlockSpec((n, d), lambda i: (0, 0)),
        ],
        out_specs=pl.BlockSpec((blk, d), lambda i: (i, 0)),
        out_shape=jax.ShapeDtypeStruct((n, d), jnp.float32),
        compiler_params=_PARALLEL,
    )(adj, y)


def _bits_to_f32(u):
    bits = jnp.where(u < 0, u ^ jnp.int32(_INT_MIN), ~u)
    return jax.lax.bitcast_convert_type(bits, jnp.float32)


def _row_topk_thresh(sim):
    blk = sim.shape[0]

    b = jax.lax.bitcast_convert_type(sim, jnp.int32)
    key = jnp.where(b < 0, ~b, b ^ jnp.int32(_INT_MIN))
    hi16 = (jax.lax.shift_right_logical(key, 16) - 32768).astype(jnp.int16)
    lo_raw = (key & jnp.int32(0xFFFF)) - 32768

    def body_hi(i, u):
        bit = jnp.left_shift(jnp.int32(1), jnp.int32(15) - i)
        cand16 = ((u | bit) - 32768).astype(jnp.int16)
        cnt = jnp.sum((hi16 >= cand16).astype(jnp.int16), axis=1,
                      keepdims=True).astype(jnp.int32)
        return jnp.where(cnt >= K1, u | bit, u)

    u = jax.lax.fori_loop(0, 16, body_hi,
                          jnp.zeros((blk, 1), jnp.int32))
    p16 = (u - 32768).astype(jnp.int16)

    cnt_above = jnp.sum((hi16 > p16).astype(jnp.int16), axis=1,
                        keepdims=True).astype(jnp.int32)
    lo16 = jnp.where(hi16 == p16, lo_raw.astype(jnp.int16),
                     jnp.int16(-32768))

    def body_lo(i, v):
        bit = jnp.left_shift(jnp.int32(1), jnp.int32(15) - i)
        cand16 = ((v | bit) - 32768).astype(jnp.int16)
        cnt = cnt_above + jnp.sum((lo16 >= cand16).astype(jnp.int16),
                                  axis=1, keepdims=True).astype(jnp.int32)
        return jnp.where(cnt >= K1, v | bit, v)

    v = jax.lax.fori_loop(0, 16, body_lo,
                          jnp.zeros((blk, 1), jnp.int32))
    return _bits_to_f32(jnp.left_shift(u, 16) | v)


def _select_kernel(e_blk_ref, et_ref, adj_ref, o_ref):
    sim = jax.lax.dot_general(
        e_blk_ref[...], et_ref[...], (((1,), (0,)), ((), ())),
        precision=_HIGH, preferred_element_type=jnp.float32)

    thresh = _row_topk_thresh(sim)

    keep = (sim >= thresh) & (sim > EPS)
    o_ref[...] = jnp.where(keep, FUSION * sim, 0.0) + (1.0 - FUSION) * adj_ref[...]


def _select(e, e_t, adj, blk):
    n, d = e.shape
    return pl.pallas_call(
        _select_kernel,
        grid=(n // blk,),
        in_specs=[
            pl.BlockSpec((blk, d), lambda i: (i, 0)),
            pl.BlockSpec((d, n), lambda i: (0, 0)),
            pl.BlockSpec((blk, n), lambda i: (i, 0)),
        ],
        out_specs=pl.BlockSpec((blk, n), lambda i: (i, 0)),
        out_shape=jax.ShapeDtypeStruct((n, n), jnp.float32),
        compiler_params=_PARALLEL,
    )(e, e_t, adj)


def kernel(features, adj, W1, b1, W2, b2):
    n, d = features.shape
    blk = min(128, n)
    y1 = _linear(features, W1, b1)
    h = _adj_gemm(adj, y1, relu=True, blk=blk)
    y2 = _linear(h, W2, b2)
    e = _adj_gemm(adj, y2, relu=False, blk=blk)
    final_adj = _select(e, e.T, adj, blk=blk)
    return e, final_adj


# speedup vs baseline: 1.3336x; 1.3336x over previous
import functools

import jax
import jax.numpy as jnp
from jax.experimental import pallas as pl
from jax.experimental.pallas import tpu as pltpu

_PARALLEL = pltpu.CompilerParams(dimension_semantics=("parallel",))

K1 = 33
EPS = 0.3
FUSION = 0.1

_HIGH = jax.lax.Precision.DEFAULT
_INT_MIN = -2147483648


def _linear_kernel(x_ref, w_ref, b_ref, o_ref):
    o_ref[...] = (
        jnp.dot(x_ref[...], w_ref[...], precision=_HIGH,
                preferred_element_type=jnp.float32)
        + b_ref[...]
    )


def _linear(x, w, b):
    n, d = x.shape
    return pl.pallas_call(
        _linear_kernel,
        out_shape=jax.ShapeDtypeStruct((n, d), jnp.float32),
    )(x, w, b.reshape(1, d))


def _adj_gemm_kernel(adj_ref, y_ref, o_ref, *, relu):
    acc = jax.lax.dot_general(
        adj_ref[...], y_ref[...], (((1,), (0,)), ((), ())),
        precision=_HIGH, preferred_element_type=jnp.float32)
    o_ref[...] = jnp.maximum(acc, 0.0) if relu else acc


def _adj_gemm(adj, y, relu, blk):
    n, d = y.shape
    return pl.pallas_call(
        functools.partial(_adj_gemm_kernel, relu=relu),
        grid=(n // blk,),
        in_specs=[
            pl.BlockSpec((blk, n), lambda i: (i, 0)),
            pl.BlockSpec((n, d), lambda i: (0, 0)),
        ],
        out_specs=pl.BlockSpec((blk, d), lambda i: (i, 0)),
        out_shape=jax.ShapeDtypeStruct((n, d), jnp.float32),
        compiler_params=_PARALLEL,
    )(adj, y)


def _bits_to_f32(u):
    bits = jnp.where(u < 0, u ^ jnp.int32(_INT_MIN), ~u)
    return jax.lax.bitcast_convert_type(bits, jnp.float32)


def _row_topk_thresh(sim):
    blk, n = sim.shape
    ones = jnp.ones((n, 1), jnp.float32)

    def body(i, t):
        bit = jnp.left_shift(jnp.int32(1), jnp.int32(31) - i)
        cand = t | bit
        cand_f = _bits_to_f32(cand)
        mask = jnp.where(sim >= cand_f, 1.0, 0.0)
        cnt = jax.lax.dot_general(
            mask, ones, (((1,), (0,)), ((), ())),
            precision=_HIGH, preferred_element_type=jnp.float32)
        return jnp.where(cnt >= float(K1), cand, t)

    t = jax.lax.fori_loop(0, 32, body, jnp.zeros((blk, 1), jnp.int32))
    return _bits_to_f32(t)


def _select_kernel(e_blk_ref, et_ref, adj_ref, o_ref):
    sim = jax.lax.dot_general(
        e_blk_ref[...], et_ref[...], (((1,), (0,)), ((), ())),
        precision=_HIGH, preferred_element_type=jnp.float32)

    thresh = _row_topk_thresh(sim)

    keep = (sim >= thresh) & (sim > EPS)
    o_ref[...] = jnp.where(keep, FUSION * sim, 0.0) + (1.0 - FUSION) * adj_ref[...]


def _select(e, e_t, adj, blk):
    n, d = e.shape
    return pl.pallas_call(
        _select_kernel,
        grid=(n // blk,),
        in_specs=[
            pl.BlockSpec((blk, d), lambda i: (i, 0)),
            pl.BlockSpec((d, n), lambda i: (0, 0)),
            pl.BlockSpec((blk, n), lambda i: (i, 0)),
        ],
        out_specs=pl.BlockSpec((blk, n), lambda i: (i, 0)),
        out_shape=jax.ShapeDtypeStruct((n, n), jnp.float32),
        compiler_params=_PARALLEL,
    )(e, e_t, adj)


def kernel(features, adj, W1, b1, W2, b2):
    n, d = features.shape
    blk = min(128, n)
    y1 = _linear(features, W1, b1)
    h = _adj_gemm(adj, y1, relu=True, blk=blk)
    y2 = _linear(h, W2, b2)
    e = _adj_gemm(adj, y2, relu=False, blk=blk)
    final_adj = _select(e, e.T, adj, blk=blk)
    return e, final_adj


# speedup vs baseline: 2.0065x; 1.5046x over previous
import functools

import jax
import jax.numpy as jnp
from jax.experimental import pallas as pl
from jax.experimental.pallas import tpu as pltpu

_PARALLEL = pltpu.CompilerParams(dimension_semantics=("parallel",))

K1 = 33
EPS = 0.3
FUSION = 0.1

_HIGH = jax.lax.Precision.DEFAULT
_INT_MIN = -2147483648


def _linear_kernel(x_ref, w_ref, b_ref, o_ref):
    o_ref[...] = (
        jnp.dot(x_ref[...], w_ref[...], precision=_HIGH,
                preferred_element_type=jnp.float32)
        + b_ref[...]
    )


def _linear(x, w, b):
    n, d = x.shape
    return pl.pallas_call(
        _linear_kernel,
        out_shape=jax.ShapeDtypeStruct((n, d), jnp.float32),
    )(x, w, b.reshape(1, d))


def _adj_gemm_kernel(adj_ref, y_ref, o_ref, *, relu):
    acc = jax.lax.dot_general(
        adj_ref[...], y_ref[...], (((1,), (0,)), ((), ())),
        precision=_HIGH, preferred_element_type=jnp.float32)
    o_ref[...] = jnp.maximum(acc, 0.0) if relu else acc


def _adj_gemm(adj, y, relu, blk):
    n, d = y.shape
    return pl.pallas_call(
        functools.partial(_adj_gemm_kernel, relu=relu),
        grid=(n // blk,),
        in_specs=[
            pl.BlockSpec((blk, n), lambda i: (i, 0)),
            pl.BlockSpec((n, d), lambda i: (0, 0)),
        ],
        out_specs=pl.BlockSpec((blk, d), lambda i: (i, 0)),
        out_shape=jax.ShapeDtypeStruct((n, d), jnp.float32),
        compiler_params=_PARALLEL,
    )(adj, y)


def _bits_to_f32(u):
    bits = jnp.where(u < 0, u ^ jnp.int32(_INT_MIN), ~u)
    return jax.lax.bitcast_convert_type(bits, jnp.float32)


def _row_topk_thresh(sim):
    blk = sim.shape[0]

    def body(i, t):
        bit = jnp.left_shift(jnp.int32(1), jnp.int32(31) - i)
        cand = t | bit
        cand_f = _bits_to_f32(cand)
        cnt = jnp.sum((sim >= cand_f).astype(jnp.float32), axis=1,
                      keepdims=True)
        return jnp.where(cnt >= float(K1), cand, t)

    t = jax.lax.fori_loop(0, 32, body, jnp.zeros((blk, 1), jnp.int32))
    return _bits_to_f32(t)


def _select_kernel(e_blk_ref, et_ref, adj_ref, o_ref):
    sim = jax.lax.dot_general(
        e_blk_ref[...], et_ref[...], (((1,), (0,)), ((), ())),
        precision=_HIGH, preferred_element_type=jnp.float32)

    thresh = _row_topk_thresh(sim)

    keep = (sim >= thresh) & (sim > EPS)
    o_ref[...] = jnp.where(keep, FUSION * sim, 0.0) + (1.0 - FUSION) * adj_ref[...]


def _select(e, e_t, adj, blk):
    n, d = e.shape
    return pl.pallas_call(
        _select_kernel,
        grid=(n // blk,),
        in_specs=[
            pl.BlockSpec((blk, d), lambda i: (i, 0)),
            pl.BlockSpec((d, n), lambda i: (0, 0)),
            pl.BlockSpec((blk, n), lambda i: (i, 0)),
        ],
        out_specs=pl.BlockSpec((blk, n), lambda i: (i, 0)),
        out_shape=jax.ShapeDtypeStruct((n, n), jnp.float32),
        compiler_params=_PARALLEL,
    )(e, e_t, adj)


def kernel(features, adj, W1, b1, W2, b2):
    n, d = features.shape
    blk = min(128, n)
    y1 = _linear(features, W1, b1)
    h = _adj_gemm(adj, y1, relu=True, blk=blk)
    y2 = _linear(h, W2, b2)
    e = _adj_gemm(adj, y2, relu=False, blk=blk)
    final_adj = _select(e, e.T, adj, blk=min(256, n))
    return e, final_adj


# speedup vs baseline: 2.0860x; 1.0396x over previous
import functools

import jax
import jax.numpy as jnp
from jax.experimental import pallas as pl
from jax.experimental.pallas import tpu as pltpu

_PARALLEL = pltpu.CompilerParams(dimension_semantics=("parallel",))

K1 = 33
EPS = 0.3
FUSION = 0.1

_HIGH = jax.lax.Precision.DEFAULT
_INT_MIN = -2147483648


def _linear_kernel(x_ref, w_ref, b_ref, o_ref):
    o_ref[...] = (
        jnp.dot(x_ref[...], w_ref[...], precision=_HIGH,
                preferred_element_type=jnp.float32)
        + b_ref[...]
    )


def _linear(x, w, b):
    n, d = x.shape
    return pl.pallas_call(
        _linear_kernel,
        out_shape=jax.ShapeDtypeStruct((n, d), jnp.float32),
    )(x, w, b.reshape(1, d))


def _adj_gemm_kernel(adj_ref, y_ref, o_ref, *, relu):
    acc = jax.lax.dot_general(
        adj_ref[...], y_ref[...], (((1,), (0,)), ((), ())),
        precision=_HIGH, preferred_element_type=jnp.float32)
    o_ref[...] = jnp.maximum(acc, 0.0) if relu else acc


def _adj_gemm(adj, y, relu, blk):
    n, d = y.shape
    return pl.pallas_call(
        functools.partial(_adj_gemm_kernel, relu=relu),
        grid=(n // blk,),
        in_specs=[
            pl.BlockSpec((blk, n), lambda i: (i, 0)),
            pl.BlockSpec((n, d), lambda i: (0, 0)),
        ],
        out_specs=pl.BlockSpec((blk, d), lambda i: (i, 0)),
        out_shape=jax.ShapeDtypeStruct((n, d), jnp.float32),
        compiler_params=_PARALLEL,
    )(adj, y)


def _bits_to_f32(u):
    bits = jnp.where(u < 0, u ^ jnp.int32(_INT_MIN), ~u)
    return jax.lax.bitcast_convert_type(bits, jnp.float32)


def _row_topk_thresh(sim):
    blk = sim.shape[0]

    def body(i, t):
        bit = jnp.left_shift(jnp.int32(1), jnp.int32(31) - i)
        cand = t | bit
        cand_f = _bits_to_f32(cand)
        cnt = jnp.sum((sim >= cand_f).astype(jnp.float32), axis=1,
                      keepdims=True)
        return jnp.where(cnt >= float(K1), cand, t)

    t = jax.lax.fori_loop(0, 32, body, jnp.zeros((blk, 1), jnp.int32))
    return _bits_to_f32(t)


def _select_kernel(e_blk_ref, et_ref, adj_ref, o_ref):
    sim = jax.lax.dot_general(
        e_blk_ref[...], et_ref[...], (((1,), (0,)), ((), ())),
        precision=_HIGH, preferred_element_type=jnp.float32)

    thresh = _row_topk_thresh(sim)

    eps_next = jnp.float32(0.30000004172325134)
    keep = sim >= jnp.maximum(thresh, eps_next)
    o_ref[...] = jnp.where(keep, FUSION * sim, 0.0) + (1.0 - FUSION) * adj_ref[...]


def _select(e, e_t, adj, blk):
    n, d = e.shape
    return pl.pallas_call(
        _select_kernel,
        grid=(n // blk,),
        in_specs=[
            pl.BlockSpec((blk, d), lambda i: (i, 0)),
            pl.BlockSpec((d, n), lambda i: (0, 0)),
            pl.BlockSpec((blk, n), lambda i: (i, 0)),
        ],
        out_specs=pl.BlockSpec((blk, n), lambda i: (i, 0)),
        out_shape=jax.ShapeDtypeStruct((n, n), jnp.float32),
        compiler_params=_PARALLEL,
    )(e, e_t, adj)


def kernel(features, adj, W1, b1, W2, b2):
    n, d = features.shape
    blk = min(256, n)
    y1 = _linear(features, W1, b1)
    h = _adj_gemm(adj, y1, relu=True, blk=blk)
    y2 = _linear(h, W2, b2)
    e = _adj_gemm(adj, y2, relu=False, blk=blk)
    final_adj = _select(e, e.T, adj, blk=min(256, n))
    return e, final_adj
